# baseline (speedup 1.0000x reference)
import functools

import jax
import jax.numpy as jnp
from jax import lax
from jax.experimental import pallas as pl
from jax.experimental.pallas import tpu as pltpu
from jax.experimental.pallas import tpu_sc as plsc

N = 10000
E = 320000
F = 128
HDIM = 128
NCLASS = 2
WIDTH = 3
K = 3
B = 64

NC = 2
NS = 16
NTILE = NC * NS
EC = 128
EPAD = 327680
EROWS = EPAD // EC
DROWS = EROWS // NTILE
SROWS = EROWS // NS
PH = 4
HROWS = SROWS // PH
NBUF = 4
FH = F // NC
ET = E // NTILE
NPAD = 10240
NPT = NPAD // NS
DPT = NPAD // NS

R = 1000
NBLK = N // R


def _leaky(x):
    return jnp.where(x >= 0, x, 0.01 * x)


def _sc_deg(dst2d, ones_h, zeros_h):
    mesh = plsc.VectorSubcoreMesh(core_axis_name="c", subcore_axis_name="s",
                                  num_cores=NC, num_subcores=NS)

    @functools.partial(
        pl.kernel,
        out_type=jax.ShapeDtypeStruct((NC, NPAD), jnp.float32),
        mesh=mesh,
        scratch_types=[
            pltpu.VMEM((DROWS, EC), jnp.int32),
            pltpu.VMEM((EC,), jnp.float32),
            pltpu.VMEM_SHARED((NPAD,), jnp.float32),
        ],
        compiler_params=pltpu.CompilerParams(use_tc_tiling_on_sc=False),
    )
    def k(dst_h, ones_hh, zeros_hh, out_h, idx_v, ones_v, acc):
        c = lax.axis_index("c")
        s = lax.axis_index("s")
        tile = c * NS + s
        pltpu.sync_copy(dst_h.at[pl.ds(tile * DROWS, DROWS)], idx_v)
        pltpu.sync_copy(ones_hh, ones_v)
        pltpu.sync_copy(zeros_hh, acc.at[pl.ds(s * DPT, DPT)])
        plsc.subcore_barrier()

        def body(j, carry):
            pltpu.sync_copy(ones_v, acc.at[idx_v.at[j]], add=True)
            return carry

        lax.fori_loop(0, DROWS, body, 0)
        plsc.subcore_barrier()
        pltpu.sync_copy(acc.at[pl.ds(s * DPT, DPT)],
                        out_h.at[c, pl.ds(s * DPT, DPT)])

    return k(dst2d, ones_h, zeros_h)


def _sc_spmm(gsp, src2d, dst2d, zrows):
    mesh = plsc.VectorSubcoreMesh(core_axis_name="c", subcore_axis_name="s",
                                  num_cores=NC, num_subcores=NS)

    @functools.partial(
        pl.kernel,
        out_type=jax.ShapeDtypeStruct((NC, NPAD, FH), jnp.float32),
        mesh=mesh,
        scratch_types=[
            pltpu.VMEM((HROWS, EC), jnp.int32),
            pltpu.VMEM((HROWS, EC), jnp.int32),
            pltpu.VMEM((NBUF, EC, FH), jnp.float32),
            pltpu.VMEM_SHARED((NPAD, FH), jnp.float32),
            pltpu.SemaphoreType.DMA,
            pltpu.SemaphoreType.DMA,
        ],
        compiler_params=pltpu.CompilerParams(use_tc_tiling_on_sc=False),
    )
    def k(g_h, src_h, dst_h, zrows_h, out_h, sidx, didx, buf, acc, gsem, ssem):
        c = lax.axis_index("c")
        s = lax.axis_index("s")
        pltpu.sync_copy(zrows_h, acc.at[pl.ds(s * NPT, NPT)])
        plsc.subcore_barrier()

        gc = g_h.at[c]

        def drain_one():
            pltpu.make_async_copy(buf.at[0], acc.at[didx.at[0]], ssem).wait()

        for h in range(PH):
            base = s * SROWS + h * HROWS
            pltpu.sync_copy(src_h.at[pl.ds(base, HROWS)], sidx)
            pltpu.sync_copy(dst_h.at[pl.ds(base, HROWS)], didx)
            pltpu.async_copy(gc.at[sidx.at[0]], buf.at[0], gsem)
            pltpu.async_copy(gc.at[sidx.at[1]], buf.at[1], gsem)
            pltpu.async_copy(gc.at[sidx.at[2]], buf.at[2], gsem)

            def body(j, carry):
                slot = lax.rem(j, NBUF)
                pltpu.make_async_copy(gc.at[sidx.at[j]], buf.at[slot],
                                      gsem).wait()
                pltpu.async_copy(buf.at[slot], acc.at[didx.at[j]], ssem,
                                 add=True)

                @pl.when(j + 3 < HROWS)
                def _():
                    @pl.when(j >= 1)
                    def _():
                        drain_one()
                    pltpu.async_copy(gc.at[sidx.at[j + 3]],
                                     buf.at[lax.rem(j + 3, NBUF)], gsem)

                return carry

            lax.fori_loop(0, HROWS, body, 0)
            drain_one()
            drain_one()
            drain_one()
            drain_one()
        plsc.subcore_barrier()
        pltpu.sync_copy(acc.at[pl.ds(s * NPT, NPT)],
                        out_h.at[c, pl.ds(s * NPT, NPT)])

    return k(gsp, src2d, dst2d, zrows)


def _dinv_from(degp_ref):
    deg = degp_ref[0, 0, :] + degp_ref[0, 1, :]
    return jnp.where(deg > 0, lax.rsqrt(jnp.maximum(deg, 1.0)), 0.0)


def _tc_pre_body(x_ref, degp_ref, xlx_ref, w1_ref, b1_ref, w2_ref, b2_ref,
                 chw_ref, chb_ref, w3_ref, b3_ref, w5_ref, b5_ref, w6_ref,
                 b6_ref, w8_ref, b8_ref, w9_ref, b9_ref,
                 h_ref, g1_ref, tmp_ref, xlxv_ref, wc_ref, bc_ref):
    i = pl.program_id(0)
    x = x_ref[...]
    h1 = _leaky(jnp.dot(x, w1_ref[...], preferred_element_type=jnp.float32)
                + b1_ref[...])
    h2 = _leaky(jnp.dot(h1, w2_ref[...], preferred_element_type=jnp.float32)
                + b2_ref[...]) + h1
    h_ref[...] = h2
    dinv = _dinv_from(degp_ref)
    g = h2 * dinv[:, None]
    g1_ref[0] = g[:, :FH]
    g1_ref[1] = g[:, FH:]

    @pl.when(i == 0)
    def _():
        xlx = xlx_ref[...]
        t = _leaky(jnp.dot(xlx, w8_ref[...],
                           preferred_element_type=jnp.float32) + b8_ref[...])
        t = _leaky(jnp.dot(t, w9_ref[...],
                           preferred_element_type=jnp.float32) + b9_ref[...])
        tmp_ref[...] = t
        v = jnp.dot(xlx, w5_ref[...],
                    preferred_element_type=jnp.float32) + b5_ref[...]
        v = jnp.dot(v, w6_ref[...],
                    preferred_element_type=jnp.float32) + b6_ref[...]
        xlxv_ref[...] = _leaky(v)
        bc = b3_ref[...]
        for kk in range(K):
            acc = jnp.zeros((F, F), dtype=jnp.float32)
            for ii in range(WIDTH):
                acc = acc + jnp.dot(chw_ref[ii, kk],
                                    w3_ref[pl.ds(ii * F, F), :],
                                    preferred_element_type=jnp.float32)
            wc_ref[kk] = acc
        for ii in range(WIDTH):
            bc = bc + jnp.dot(chb_ref[ii], w3_ref[pl.ds(ii * F, F), :],
                              preferred_element_type=jnp.float32)
        bc_ref[...] = bc


def _tc_pre(x, degp, xlx, w1, b1, w2, b2, chw, chb, w3, b3, w5, b5, w6, b6,
            w8, b8, w9, b9):
    full = lambda shape: pl.BlockSpec(shape, lambda i: tuple(0 for _ in shape))
    out_shapes = (
        jax.ShapeDtypeStruct((N, F), jnp.float32),
        jax.ShapeDtypeStruct((NC, N, FH), jnp.float32),
        jax.ShapeDtypeStruct((B, HDIM), jnp.float32),
        jax.ShapeDtypeStruct((B, HDIM), jnp.float32),
        jax.ShapeDtypeStruct((K, F, F), jnp.float32),
        jax.ShapeDtypeStruct((1, F), jnp.float32),
    )
    grid = (NBLK,)
    return pl.pallas_call(
        _tc_pre_body,
        grid=grid,
        in_specs=[
            pl.BlockSpec((R, F), lambda i: (i, 0)),
            pl.BlockSpec((1, NC, R), lambda i: (i, 0, 0)),
            full((B, F)),
            full((F, F)), full((1, F)), full((F, F)), full((1, F)),
            full((WIDTH, K, F, F)), full((WIDTH, 1, F)),
            full((WIDTH * F, HDIM)), full((1, HDIM)),
            full((F, HDIM)), full((1, HDIM)), full((HDIM, HDIM)),
            full((1, HDIM)),
            full((F, HDIM)), full((1, HDIM)), full((HDIM, HDIM)),
            full((1, HDIM)),
        ],
        out_specs=(
            pl.BlockSpec((R, F), lambda i: (i, 0)),
            pl.BlockSpec((NC, R, FH), lambda i: (0, i, 0)),
            full((B, HDIM)),
            full((B, HDIM)),
            full((K, F, F)),
            full((1, F)),
        ),
        out_shape=out_shapes,
        compiler_params=pltpu.CompilerParams(
            dimension_semantics=("arbitrary",)),
    )(x, degp, xlx, w1, b1, w2, b2, chw, chb, w3, b3, w5, b5, w6, b6,
      w8, b8, w9, b9)


def _tc_mid_body(p1_ref, degp_ref, tx1_ref, g2_ref):
    dinv = _dinv_from(degp_ref)
    s1 = jnp.concatenate([p1_ref[0], p1_ref[1]], axis=-1)
    tx1 = -dinv[:, None] * s1
    tx1_ref[...] = tx1
    g2 = dinv[:, None] * tx1
    g2_ref[0] = g2[:, :FH]
    g2_ref[1] = g2[:, FH:]


def _tc_mid(p1, degp):
    return pl.pallas_call(
        _tc_mid_body,
        grid=(NBLK,),
        in_specs=[
            pl.BlockSpec((NC, R, FH), lambda i: (0, i, 0)),
            pl.BlockSpec((1, NC, R), lambda i: (i, 0, 0)),
        ],
        out_specs=(
            pl.BlockSpec((R, F), lambda i: (i, 0)),
            pl.BlockSpec((NC, R, FH), lambda i: (0, i, 0)),
        ),
        out_shape=(
            jax.ShapeDtypeStruct((N, F), jnp.float32),
            jax.ShapeDtypeStruct((NC, N, FH), jnp.float32),
        ),
        compiler_params=pltpu.CompilerParams(
            dimension_semantics=("arbitrary",)),
    )(p1, degp)


def _tc_fin_body(h_ref, tx1_ref, p2_ref, degp_ref, n2g_ref, wc_ref, bc_ref,
                 w4_ref, b4_ref, tmp_ref, xlxv_ref, w7p_ref, b7p_ref,
                 accA_ref, accC_ref, emb_ref):
    i = pl.program_id(0)
    dinv = _dinv_from(degp_ref)
    h = h_ref[...]
    s2 = jnp.concatenate([p2_ref[0], p2_ref[1]], axis=-1)
    tx2 = -2.0 * dinv[:, None] * s2 - h
    pre = (jnp.dot(h, wc_ref[0], preferred_element_type=jnp.float32)
           + jnp.dot(tx1_ref[...], wc_ref[1],
                     preferred_element_type=jnp.float32)
           + jnp.dot(tx2, wc_ref[2], preferred_element_type=jnp.float32)
           + bc_ref[...])
    hh = _leaky(pre)
    hh = _leaky(jnp.dot(hh, w4_ref[...], preferred_element_type=jnp.float32)
                + b4_ref[...])
    n2g = n2g_ref[0, 0, :]
    onehot = (n2g[:, None] == lax.broadcasted_iota(jnp.int32, (R, B), 1)
              ).astype(jnp.float32)
    tmpn = jnp.dot(onehot, tmp_ref[...], preferred_element_type=jnp.float32)
    scores = jnp.sum(hh * tmpn, axis=1, keepdims=True)
    sh = scores * hh
    contribA = lax.dot_general(onehot, sh, (((0,), (0,)), ((), ())),
                               preferred_element_type=jnp.float32)
    ones_mat = jnp.ones((R, F), dtype=jnp.float32)
    contribC = lax.dot_general(onehot, ones_mat, (((0,), (0,)), ((), ())),
                               preferred_element_type=jnp.float32)

    @pl.when(i == 0)
    def _():
        accA_ref[...] = jnp.zeros_like(accA_ref)
        accC_ref[...] = jnp.zeros_like(accC_ref)

    accA_ref[...] += contribA
    accC_ref[...] += contribC

    @pl.when(i == NBLK - 1)
    def _():
        pooled = accA_ref[...] / jnp.maximum(accC_ref[...], 1.0)
        emb = (jnp.dot(pooled, w7p_ref[pl.ds(0, HDIM), :],
                       preferred_element_type=jnp.float32)
               + jnp.dot(xlxv_ref[...], w7p_ref[pl.ds(HDIM, HDIM), :],
                         preferred_element_type=jnp.float32)
               + b7p_ref[...])
        emb_ref[...] = emb


def _tc_fin(h, tx1, p2, degp, n2g3, wc, bc, w4, b4, tmp, xlxv, w7p, b7p):
    full = lambda shape: pl.BlockSpec(shape, lambda i: tuple(0 for _ in shape))
    return pl.pallas_call(
        _tc_fin_body,
        grid=(NBLK,),
        in_specs=[
            pl.BlockSpec((R, F), lambda i: (i, 0)),
            pl.BlockSpec((R, F), lambda i: (i, 0)),
            pl.BlockSpec((NC, R, FH), lambda i: (0, i, 0)),
            pl.BlockSpec((1, NC, R), lambda i: (i, 0, 0)),
            pl.BlockSpec((1, 1, R), lambda i: (i, 0, 0)),
            full((K, F, F)), full((1, F)),
            full((HDIM, HDIM)), full((1, HDIM)),
            full((B, HDIM)), full((B, HDIM)),
            full((2 * HDIM, 128)), full((1, 128)),
        ],
        out_specs=(
            full((B, HDIM)),
            full((B, HDIM)),
            full((B, 128)),
        ),
        out_shape=(
            jax.ShapeDtypeStruct((B, HDIM), jnp.float32),
            jax.ShapeDtypeStruct((B, HDIM), jnp.float32),
            jax.ShapeDtypeStruct((B, 128), jnp.float32),
        ),
        compiler_params=pltpu.CompilerParams(
            dimension_semantics=("arbitrary",)),
    )(h, tx1, p2, degp, n2g3, wc, bc, w4, b4, tmp, xlxv, w7p, b7p)


def kernel(features_list, xLx_batch, edge_index, node2graph, W1, b1, W2, b2,
           cheb_W, cheb_b, W3, b3, W4, b4, W5, b5, W6, b6, W7, b7, W8, b8,
           W9, b9):
    src2d = jnp.concatenate(
        [edge_index[0], jnp.zeros((EPAD - E,), jnp.int32)]
    ).reshape(EROWS, EC)
    dst2d = jnp.concatenate(
        [edge_index[1], jnp.full((EPAD - E,), N, jnp.int32)]
    ).reshape(EROWS, EC)
    n2g3 = node2graph.reshape(NBLK, 1, R)
    ones_ec = jnp.ones((EC,), dtype=jnp.float32)
    zeros_dpt = jnp.zeros((DPT,), dtype=jnp.float32)
    zrows = jnp.zeros((NPT, FH), dtype=jnp.float32)
    w7p = jnp.pad(W7, ((0, 0), (0, 128 - NCLASS)))
    b7p = jnp.pad(b7, (0, 128 - NCLASS)).reshape(1, 128)
    r2 = lambda v: v.reshape(1, -1)

    degp = _sc_deg(dst2d, ones_ec, zeros_dpt)
    degp = degp[:, :N].reshape(NC, NBLK, R).transpose(1, 0, 2)

    h, g1, tmp, xlxv, wc, bc = _tc_pre(
        features_list, degp, xLx_batch, W1, r2(b1), W2, r2(b2),
        cheb_W, cheb_b.reshape(WIDTH, 1, F), W3, r2(b3), W5, r2(b5),
        W6, r2(b6), W8, r2(b8), W9, r2(b9))

    p1 = _sc_spmm(g1, src2d, dst2d, zrows)
    tx1, g2 = _tc_mid(p1, degp)
    p2 = _sc_spmm(g2, src2d, dst2d, zrows)

    _, _, emb = _tc_fin(h, tx1, p2, degp, n2g3, wc, bc, W4, r2(b4),
                        tmp, xlxv, w7p, b7p)
    return emb[:, :NCLASS]


# speedup vs baseline: 1.0590x; 1.0590x over previous
import functools

import jax
import jax.numpy as jnp
from jax import lax
from jax.experimental import pallas as pl
from jax.experimental.pallas import tpu as pltpu
from jax.experimental.pallas import tpu_sc as plsc

N = 10000
E = 320000
F = 128
HDIM = 128
NCLASS = 2
WIDTH = 3
K = 3
B = 64

NC = 2
NS = 16
NTILE = NC * NS
EC = 128
EPAD = 327680
EROWS = EPAD // EC
DROWS = EROWS // NTILE
SROWS = EROWS // NS
PH = 2
HROWS = SROWS // PH
NBUF = 3
RCH = 5
FH = F // NC
ET = E // NTILE
NPAD = 10240
NPT = NPAD // NS
DPT = NPAD // NS

R = 1000
NBLK = N // R


def _leaky(x):
    return jnp.where(x >= 0, x, 0.01 * x)


def _sc_deg(dst2d, ones_h, zeros_h):
    mesh = plsc.VectorSubcoreMesh(core_axis_name="c", subcore_axis_name="s",
                                  num_cores=NC, num_subcores=NS)

    @functools.partial(
        pl.kernel,
        out_type=jax.ShapeDtypeStruct((NC, NPAD), jnp.float32),
        mesh=mesh,
        scratch_types=[
            pltpu.VMEM((DROWS, EC), jnp.int32),
            pltpu.VMEM((EC,), jnp.float32),
            pltpu.VMEM_SHARED((NPAD,), jnp.float32),
        ],
        compiler_params=pltpu.CompilerParams(use_tc_tiling_on_sc=False),
    )
    def k(dst_h, ones_hh, zeros_hh, out_h, idx_v, ones_v, acc):
        c = lax.axis_index("c")
        s = lax.axis_index("s")
        tile = c * NS + s
        pltpu.sync_copy(dst_h.at[pl.ds(tile * DROWS, DROWS)], idx_v)
        pltpu.sync_copy(ones_hh, ones_v)
        pltpu.sync_copy(zeros_hh, acc.at[pl.ds(s * DPT, DPT)])
        plsc.subcore_barrier()

        def body(j, carry):
            pltpu.sync_copy(ones_v, acc.at[idx_v.at[j]], add=True)
            return carry

        lax.fori_loop(0, DROWS, body, 0)
        plsc.subcore_barrier()
        pltpu.sync_copy(acc.at[pl.ds(s * DPT, DPT)],
                        out_h.at[c, pl.ds(s * DPT, DPT)])

    return k(dst2d, ones_h, zeros_h)


def _sc_spmm(gsp, src2d, dst2d, zrows):
    mesh = plsc.VectorSubcoreMesh(core_axis_name="c", subcore_axis_name="s",
                                  num_cores=NC, num_subcores=NS)

    @functools.partial(
        pl.kernel,
        out_type=jax.ShapeDtypeStruct((NC, NPAD, FH), jnp.float32),
        mesh=mesh,
        scratch_types=[
            pltpu.VMEM((HROWS, EC), jnp.int32),
            pltpu.VMEM((HROWS, EC), jnp.int32),
            pltpu.VMEM((NBUF, EC, FH), jnp.float32),
            pltpu.VMEM_SHARED((NPAD, FH), jnp.float32),
            pltpu.SemaphoreType.DMA,
            pltpu.SemaphoreType.DMA,
        ],
        compiler_params=pltpu.CompilerParams(use_tc_tiling_on_sc=False),
    )
    def k(g_h, src_h, dst_h, zrows_h, out_h, sidx, didx, buf, acc, gsem, ssem):
        c = lax.axis_index("c")
        s = lax.axis_index("s")
        pltpu.sync_copy(zrows_h, acc.at[pl.ds(s * NPT, NPT)])
        plsc.subcore_barrier()

        gc = g_h.at[c]

        def drain_one():
            pltpu.make_async_copy(buf.at[0], acc.at[didx.at[0]], ssem).wait()

        for h in range(PH):
            base = s * SROWS + h * HROWS
            pltpu.sync_copy(src_h.at[pl.ds(base, HROWS)], sidx)
            pltpu.sync_copy(dst_h.at[pl.ds(base, HROWS)], didx)
            pltpu.async_copy(gc.at[sidx.at[0]], buf.at[0], gsem)
            pltpu.async_copy(gc.at[sidx.at[1]], buf.at[1], gsem)

            def body(j, carry):
                slot = lax.rem(j, NBUF)
                pltpu.make_async_copy(gc.at[sidx.at[j]], buf.at[slot],
                                      gsem).wait()
                pltpu.async_copy(buf.at[slot], acc.at[didx.at[j]], ssem,
                                 add=True)

                @pl.when(j + 2 < HROWS)
                def _():
                    @pl.when(j >= 1)
                    def _():
                        drain_one()
                    pltpu.async_copy(gc.at[sidx.at[j + 2]],
                                     buf.at[lax.rem(j + 2, NBUF)], gsem)

                return carry

            lax.fori_loop(0, HROWS, body, 0)
            drain_one()
            drain_one()
            drain_one()
        plsc.subcore_barrier()
        pltpu.sync_copy(acc.at[pl.ds(s * NPT, NPT)],
                        out_h.at[c, pl.ds(s * NPT, NPT)])

    return k(gsp, src2d, dst2d, zrows)


def _dinv_from(degp_ref):
    deg = degp_ref[0, 0, :] + degp_ref[0, 1, :]
    return jnp.where(deg > 0, lax.rsqrt(jnp.maximum(deg, 1.0)), 0.0)


def _tc_pre_body(x_ref, degp_ref, xlx_ref, w1_ref, b1_ref, w2_ref,
                 b2_ref, chw_ref, chb_ref, w3_ref, b3_ref, w5_ref, b5_ref,
                 w6_ref, b6_ref, w8_ref, b8_ref, w9_ref, b9_ref,
                 h_ref, g1_ref, tmp_ref, xlxv_ref, wc_ref, bc_ref):
    i = pl.program_id(0)
    x = x_ref[...]
    h1 = _leaky(jnp.dot(x, w1_ref[...], preferred_element_type=jnp.float32)
                + b1_ref[...])
    h2 = _leaky(jnp.dot(h1, w2_ref[...], preferred_element_type=jnp.float32)
                + b2_ref[...]) + h1
    h_ref[...] = h2
    dinv = _dinv_from(degp_ref)
    g = h2 * dinv[:, None]
    g1_ref[0] = g[:, :FH]
    g1_ref[1] = g[:, FH:]

    @pl.when(i == 0)
    def _():
        xlx = xlx_ref[...]
        t = _leaky(jnp.dot(xlx, w8_ref[...],
                           preferred_element_type=jnp.float32) + b8_ref[...])
        t = _leaky(jnp.dot(t, w9_ref[...],
                           preferred_element_type=jnp.float32) + b9_ref[...])
        tmp_ref[...] = t
        v = jnp.dot(xlx, w5_ref[...],
                    preferred_element_type=jnp.float32) + b5_ref[...]
        v = jnp.dot(v, w6_ref[...],
                    preferred_element_type=jnp.float32) + b6_ref[...]
        xlxv_ref[...] = _leaky(v)
        bc = b3_ref[...]
        for kk in range(K):
            acc = jnp.zeros((F, F), dtype=jnp.float32)
            for ii in range(WIDTH):
                acc = acc + jnp.dot(chw_ref[ii, kk],
                                    w3_ref[pl.ds(ii * F, F), :],
                                    preferred_element_type=jnp.float32)
            wc_ref[kk] = acc
        for ii in range(WIDTH):
            bc = bc + jnp.dot(chb_ref[ii], w3_ref[pl.ds(ii * F, F), :],
                              preferred_element_type=jnp.float32)
        bc_ref[...] = bc


def _tc_pre(x, degp, xlx, w1, b1, w2, b2, chw, chb, w3, b3, w5, b5,
            w6, b6, w8, b8, w9, b9):
    full = lambda shape: pl.BlockSpec(shape, lambda i: tuple(0 for _ in shape))
    out_shapes = (
        jax.ShapeDtypeStruct((N, F), jnp.float32),
        jax.ShapeDtypeStruct((NC, N, FH), jnp.float32),
        jax.ShapeDtypeStruct((B, HDIM), jnp.float32),
        jax.ShapeDtypeStruct((B, HDIM), jnp.float32),
        jax.ShapeDtypeStruct((K, F, F), jnp.float32),
        jax.ShapeDtypeStruct((1, F), jnp.float32),
    )
    grid = (NBLK,)
    return pl.pallas_call(
        _tc_pre_body,
        grid=grid,
        in_specs=[
            pl.BlockSpec((R, F), lambda i: (i, 0)),
            pl.BlockSpec((1, NC, R), lambda i: (i, 0, 0)),
            full((B, F)),
            full((F, F)), full((1, F)), full((F, F)), full((1, F)),
            full((WIDTH, K, F, F)), full((WIDTH, 1, F)),
            full((WIDTH * F, HDIM)), full((1, HDIM)),
            full((F, HDIM)), full((1, HDIM)), full((HDIM, HDIM)),
            full((1, HDIM)),
            full((F, HDIM)), full((1, HDIM)), full((HDIM, HDIM)),
            full((1, HDIM)),
        ],
        out_specs=(
            pl.BlockSpec((R, F), lambda i: (i, 0)),
            pl.BlockSpec((NC, R, FH), lambda i: (0, i, 0)),
            full((B, HDIM)),
            full((B, HDIM)),
            full((K, F, F)),
            full((1, F)),
        ),
        out_shape=out_shapes,
        compiler_params=pltpu.CompilerParams(
            dimension_semantics=("arbitrary",)),
    )(x, degp, xlx, w1, b1, w2, b2, chw, chb, w3, b3, w5, b5, w6, b6,
      w8, b8, w9, b9)


def _tc_mid_body(p1_ref, degp_ref, g2_ref):
    dinv = _dinv_from(degp_ref)
    s1 = jnp.concatenate([p1_ref[0], p1_ref[1]], axis=-1)
    g2 = (-dinv * dinv)[:, None] * s1
    g2_ref[0] = g2[:, :FH]
    g2_ref[1] = g2[:, FH:]


def _tc_mid(p1, degp):
    return pl.pallas_call(
        _tc_mid_body,
        grid=(NBLK,),
        in_specs=[
            pl.BlockSpec((NC, R, FH), lambda i: (0, i, 0)),
            pl.BlockSpec((1, NC, R), lambda i: (i, 0, 0)),
        ],
        out_specs=pl.BlockSpec((NC, R, FH), lambda i: (0, i, 0)),
        out_shape=jax.ShapeDtypeStruct((NC, N, FH), jnp.float32),
        compiler_params=pltpu.CompilerParams(
            dimension_semantics=("arbitrary",)),
    )(p1, degp)


def _tc_fin_body(h_ref, g2_ref, p2_ref, degp_ref, n2g_ref, wc_ref, bc_ref,
                 w4_ref, b4_ref, tmp_ref, xlxv_ref, w7p_ref, b7p_ref,
                 accA_ref, accC_ref, emb_ref):
    i = pl.program_id(0)
    deg = degp_ref[0, 0, :] + degp_ref[0, 1, :]
    dinv = jnp.where(deg > 0, lax.rsqrt(jnp.maximum(deg, 1.0)), 0.0)
    h = h_ref[...]
    sqdeg = jnp.sqrt(jnp.maximum(deg, 1.0))
    g2 = jnp.concatenate([g2_ref[0], g2_ref[1]], axis=-1)
    tx1 = g2 * sqdeg[:, None]
    s2 = jnp.concatenate([p2_ref[0], p2_ref[1]], axis=-1)
    tx2 = -2.0 * dinv[:, None] * s2 - h
    pre = (jnp.dot(h, wc_ref[0], preferred_element_type=jnp.float32)
           + jnp.dot(tx1, wc_ref[1], preferred_element_type=jnp.float32)
           + jnp.dot(tx2, wc_ref[2], preferred_element_type=jnp.float32)
           + bc_ref[...])
    hh = _leaky(pre)
    hh = _leaky(jnp.dot(hh, w4_ref[...], preferred_element_type=jnp.float32)
                + b4_ref[...])
    n2g = n2g_ref[0, 0, :]
    onehot = (n2g[:, None] == lax.broadcasted_iota(jnp.int32, (R, B), 1)
              ).astype(jnp.float32)
    tmpn = jnp.dot(onehot, tmp_ref[...], preferred_element_type=jnp.float32)
    scores = jnp.sum(hh * tmpn, axis=1, keepdims=True)
    sh = scores * hh
    contribA = lax.dot_general(onehot, sh, (((0,), (0,)), ((), ())),
                               preferred_element_type=jnp.float32)
    ones_mat = jnp.ones((R, F), dtype=jnp.float32)
    contribC = lax.dot_general(onehot, ones_mat, (((0,), (0,)), ((), ())),
                               preferred_element_type=jnp.float32)

    @pl.when(i == 0)
    def _():
        accA_ref[...] = jnp.zeros_like(accA_ref)
        accC_ref[...] = jnp.zeros_like(accC_ref)

    accA_ref[...] += contribA
    accC_ref[...] += contribC

    @pl.when(i == NBLK - 1)
    def _():
        pooled = accA_ref[...] / jnp.maximum(accC_ref[...], 1.0)
        emb = (jnp.dot(pooled, w7p_ref[pl.ds(0, HDIM), :],
                       preferred_element_type=jnp.float32)
               + jnp.dot(xlxv_ref[...], w7p_ref[pl.ds(HDIM, HDIM), :],
                         preferred_element_type=jnp.float32)
               + b7p_ref[...])
        emb_ref[...] = emb


def _tc_fin(h, g2p, p2, degp, n2g3, wc, bc, w4, b4, tmp, xlxv, w7p, b7p):
    full = lambda shape: pl.BlockSpec(shape, lambda i: tuple(0 for _ in shape))
    return pl.pallas_call(
        _tc_fin_body,
        grid=(NBLK,),
        in_specs=[
            pl.BlockSpec((R, F), lambda i: (i, 0)),
            pl.BlockSpec((NC, R, FH), lambda i: (0, i, 0)),
            pl.BlockSpec((NC, R, FH), lambda i: (0, i, 0)),
            pl.BlockSpec((1, NC, R), lambda i: (i, 0, 0)),
            pl.BlockSpec((1, 1, R), lambda i: (i, 0, 0)),
            full((K, F, F)), full((1, F)),
            full((HDIM, HDIM)), full((1, HDIM)),
            full((B, HDIM)), full((B, HDIM)),
            full((2 * HDIM, 128)), full((1, 128)),
        ],
        out_specs=(
            full((B, HDIM)),
            full((B, HDIM)),
            full((B, 128)),
        ),
        out_shape=(
            jax.ShapeDtypeStruct((B, HDIM), jnp.float32),
            jax.ShapeDtypeStruct((B, HDIM), jnp.float32),
            jax.ShapeDtypeStruct((B, 128), jnp.float32),
        ),
        compiler_params=pltpu.CompilerParams(
            dimension_semantics=("arbitrary",)),
    )(h, g2p, p2, degp, n2g3, wc, bc, w4, b4, tmp, xlxv, w7p, b7p)


def kernel(features_list, xLx_batch, edge_index, node2graph, W1, b1, W2, b2,
           cheb_W, cheb_b, W3, b3, W4, b4, W5, b5, W6, b6, W7, b7, W8, b8,
           W9, b9):
    src2d = jnp.concatenate(
        [edge_index[0], jnp.zeros((EPAD - E,), jnp.int32)]
    ).reshape(EROWS, EC)
    dst2d = jnp.concatenate(
        [edge_index[1], jnp.full((EPAD - E,), N, jnp.int32)]
    ).reshape(EROWS, EC)
    n2g3 = node2graph.reshape(NBLK, 1, R)
    ones_ec = jnp.ones((EC,), dtype=jnp.float32)
    zeros_dpt = jnp.zeros((DPT,), dtype=jnp.float32)
    zrows = jnp.zeros((NPT, FH), dtype=jnp.float32)
    w7p = jnp.pad(W7, ((0, 0), (0, 128 - NCLASS)))
    b7p = jnp.pad(b7, (0, 128 - NCLASS)).reshape(1, 128)
    r2 = lambda v: v.reshape(1, -1)

    degp_raw = _sc_deg(dst2d, ones_ec, zeros_dpt)
    degp = degp_raw[:, :N].reshape(NC, NBLK, R).transpose(1, 0, 2)

    h, g1, tmp, xlxv, wc, bc = _tc_pre(
        features_list, degp, xLx_batch, W1, r2(b1), W2, r2(b2),
        cheb_W, cheb_b.reshape(WIDTH, 1, F), W3, r2(b3), W5, r2(b5),
        W6, r2(b6), W8, r2(b8), W9, r2(b9))

    p1 = _sc_spmm(g1, src2d, dst2d, zrows)
    g2p = _tc_mid(p1, degp)
    p2 = _sc_spmm(g2p, src2d, dst2d, zrows)

    _, _, emb = _tc_fin(h, g2p, p2, degp, n2g3, wc, bc, W4, r2(b4),
                        tmp, xlxv, w7p, b7p)
    return emb[:, :NCLASS]
